# baseline (speedup 1.0000x reference)
import functools

import jax
import jax.numpy as jnp
from jax import lax
from jax.experimental import pallas as pl
from jax.experimental.pallas import tpu as pltpu
from jax.experimental.pallas import tpu_sc as plsc

N_NODES = 10000
N_EDGES = 320000
D = 128

_ROWS = 2000


def _mlp1_body(x_ref, w_ref, b_ref, o_ref):
    y = lax.dot_general(x_ref[...], w_ref[...], (((1,), (1,)), ((), ())),
                        preferred_element_type=jnp.float32)
    o_ref[...] = jnp.maximum(y + b_ref[...], 0.0)


def _mlp1(feats, W_f, b_f):
    return pl.pallas_call(
        _mlp1_body,
        grid=(N_NODES // _ROWS,),
        in_specs=[
            pl.BlockSpec((_ROWS, D), lambda i: (i, 0)),
            pl.BlockSpec((D, D), lambda i: (0, 0)),
            pl.BlockSpec((1, D), lambda i: (0, 0)),
        ],
        out_specs=pl.BlockSpec((_ROWS, D), lambda i: (i, 0)),
        out_shape=jax.ShapeDtypeStruct((N_NODES, D), jnp.float32),
    )(feats, W_f, b_f.reshape(1, D))


_NC = 2
_NS = 16
_NW = _NC * _NS
_EB = 128
_BPW = 78
_GB = 6
_GW = _GB * _EB
_NGRP = _BPW // _GB
_NPAIR = 6
_TAILW = _NW - 4
_TAILB = _NW * _BPW
_RPT = 624
_ZR = 8


def _sc_agg(h, ei):
    mesh = plsc.VectorSubcoreMesh(core_axis_name="c", subcore_axis_name="s")

    @functools.partial(
        pl.kernel,
        out_type=jax.ShapeDtypeStruct((_NC, N_NODES, D), jnp.float32),
        mesh=mesh,
        scratch_types=[
            pltpu.VMEM((2, _GW), jnp.int32),
            pltpu.VMEM((2, _GW), jnp.int32),
            pltpu.VMEM((8, _EB), jnp.int32),
            pltpu.VMEM((8, _EB), jnp.int32),
            pltpu.VMEM((_EB, D), jnp.float32),
            pltpu.VMEM((_EB, D), jnp.float32),
            pltpu.VMEM((_ZR, D), jnp.float32),
            pltpu.VMEM_SHARED((N_NODES, D), jnp.float32),
            pltpu.SemaphoreType.DMA,
            pltpu.SemaphoreType.DMA,
            pltpu.SemaphoreType.DMA,
            pltpu.SemaphoreType.DMA,
        ],
    )
    def k(h_hbm, ei_hbm, out_hbm, sgA, sgB, dgA, dgB, rows0_v, rows1_v,
          z_v, acc_s, semA, semB, sem0, sem1):
        c_ax = lax.axis_index("c")
        s = lax.axis_index("s")
        wid = s * _NC + c_ax
        base_e = wid * (_BPW * _EB)
        rows = (rows0_v, rows1_v)
        sems = (sem0, sem1)

        def idx_start(g, sg, sem):
            off = base_e + g * _GW
            pltpu.async_copy(ei_hbm.at[pl.ds(0, 2), pl.ds(off, _GW)], sg, sem)

        def idx_wait(sg, sem):
            pltpu.make_async_copy(ei_hbm.at[pl.ds(0, 2), pl.ds(0, _GW)], sg,
                                  sem).wait()

        def g_start(sg, cc, rbuf, sem):
            pltpu.async_copy(h_hbm.at[sg.at[0, pl.ds(cc * _EB, _EB)]], rbuf,
                             sem)

        def g_wait(sg, cc, rbuf, sem):
            pltpu.make_async_copy(h_hbm.at[sg.at[0, pl.ds(cc * _EB, _EB)]],
                                  rbuf, sem).wait()

        def fill_dg(sg, dg):
            for cc in range(_GB):
                for v in range(_EB // 16):
                    dg[cc, pl.ds(v * 16, 16)] = sg[1,
                                                   pl.ds(cc * _EB + v * 16,
                                                         16)]

        def do_group(sg, dg, chain_fn):
            fill_dg(sg, dg)
            for cc in range(_GB):
                p = cc % 2
                q = (cc + 1) % 2
                if cc + 1 < _GB:
                    g_start(sg, cc + 1, rows[q], sems[q])
                else:
                    chain_fn()
                g_wait(sg, cc, rows[p], sems[p])
                pltpu.sync_copy(rows[p], acc_s.at[dg.at[cc]], add=True)

        idx_start(0, sgA, semA)
        idx_start(1, sgB, semB)

        zv = jnp.zeros((16,), jnp.float32)
        for i in range(_ZR):
            for j in range(D // 16):
                z_v[i, pl.ds(j * 16, 16)] = zv

        def zbody(r, carry):
            pltpu.sync_copy(z_v, acc_s.at[pl.ds(s * _RPT + r * _ZR, _ZR)])
            return carry

        lax.fori_loop(0, _RPT // _ZR, zbody, 0)

        @pl.when(s == _NS - 1)
        def _init_tail():
            pltpu.sync_copy(z_v, acc_s.at[pl.ds(_NS * _RPT, _ZR)])
            pltpu.sync_copy(z_v, acc_s.at[pl.ds(_NS * _RPT + _ZR, _ZR)])

        idx_wait(sgA, semA)
        g_start(sgA, 0, rows0_v, sem0)

        plsc.subcore_barrier()

        def chainB():
            idx_wait(sgB, semB)
            g_start(sgB, 0, rows0_v, sem0)

        def chainA():
            idx_wait(sgA, semA)
            g_start(sgA, 0, rows0_v, sem0)

        def body(jj, carry):
            do_group(sgA, dgA, chainB)
            idx_start(2 * jj + 2, sgA, semA)
            do_group(sgB, dgB, chainA)

            @pl.when(jj < _NPAIR - 1)
            def _prefetch_b():
                idx_start(2 * jj + 3, sgB, semB)

            return carry

        lax.fori_loop(0, _NPAIR, body, 0)

        do_group(sgA, dgA, lambda: None)

        @pl.when(wid >= _TAILW)
        def _tail_block():
            off = (_TAILB + wid - _TAILW) * _EB
            pltpu.sync_copy(ei_hbm.at[pl.ds(0, 2), pl.ds(off, _EB)],
                            sgA.at[pl.ds(0, 2), pl.ds(0, _EB)])
            for v in range(_EB // 16):
                dgA[0, pl.ds(v * 16, 16)] = sgA[1, pl.ds(v * 16, 16)]
            pltpu.async_copy(h_hbm.at[sgA.at[0, pl.ds(0, _EB)]], rows0_v,
                             sem0).wait()
            pltpu.sync_copy(rows0_v, acc_s.at[dgA.at[0]], add=True)

        plsc.subcore_barrier()

        pltpu.sync_copy(acc_s.at[pl.ds(s * _RPT, _RPT)],
                        out_hbm.at[c_ax, pl.ds(s * _RPT, _RPT)])

        @pl.when(s == _NS - 1)
        def _out_tail():
            pltpu.sync_copy(acc_s.at[pl.ds(_NS * _RPT, 2 * _ZR)],
                            out_hbm.at[c_ax, pl.ds(_NS * _RPT, 2 * _ZR)])

    return k(h, ei)


def _mlp2_body(h_ref, n0_ref, n1_ref, w_ref, b_ref, eps_ref, o_ref):
    t = 1.0 + eps_ref[0, 0] * h_ref[...] + n0_ref[0] + n1_ref[0]
    y = lax.dot_general(t, w_ref[...], (((1,), (1,)), ((), ())),
                        preferred_element_type=jnp.float32)
    o_ref[...] = jnp.maximum(y + b_ref[...], 0.0)


def _mlp2(h, n_parts, W_phy, b_phy, eps):
    blk = lambda i: (i, 0)
    return pl.pallas_call(
        _mlp2_body,
        grid=(N_NODES // _ROWS,),
        in_specs=[
            pl.BlockSpec((_ROWS, D), blk),
            pl.BlockSpec((1, _ROWS, D), lambda i: (0, i, 0)),
            pl.BlockSpec((1, _ROWS, D), lambda i: (1, i, 0)),
            pl.BlockSpec((D, D), lambda i: (0, 0)),
            pl.BlockSpec((1, D), lambda i: (0, 0)),
            pl.BlockSpec((1, 1), lambda i: (0, 0)),
        ],
        out_specs=pl.BlockSpec((_ROWS, D), blk),
        out_shape=jax.ShapeDtypeStruct((N_NODES, D), jnp.float32),
    )(h, n_parts, n_parts, W_phy, b_phy.reshape(1, D), eps.reshape(1, 1))


def kernel(feats, edge_index, W_f, b_f, W_phy, b_phy, eps):
    ei = edge_index.astype(jnp.int32)
    h = _mlp1(feats, W_f, b_f)
    n_parts = _sc_agg(h, ei)
    return _mlp2(h, n_parts, W_phy, b_phy, eps)


# speedup vs baseline: 1.0140x; 1.0140x over previous
import functools

import jax
import jax.numpy as jnp
from jax import lax
from jax.experimental import pallas as pl
from jax.experimental.pallas import tpu as pltpu
from jax.experimental.pallas import tpu_sc as plsc

N_NODES = 10000
N_EDGES = 320000
D = 128

_ROWS = 2000


def _mlp1_body(x_ref, w_ref, b_ref, o_ref):
    y = lax.dot_general(x_ref[...], w_ref[...], (((1,), (1,)), ((), ())),
                        preferred_element_type=jnp.float32)
    o_ref[...] = jnp.maximum(y + b_ref[...], 0.0)


def _mlp1(feats, W_f, b_f):
    return pl.pallas_call(
        _mlp1_body,
        grid=(N_NODES // _ROWS,),
        in_specs=[
            pl.BlockSpec((_ROWS, D), lambda i: (i, 0)),
            pl.BlockSpec((D, D), lambda i: (0, 0)),
            pl.BlockSpec((1, D), lambda i: (0, 0)),
        ],
        out_specs=pl.BlockSpec((_ROWS, D), lambda i: (i, 0)),
        out_shape=jax.ShapeDtypeStruct((N_NODES, D), jnp.float32),
    )(feats, W_f, b_f.reshape(1, D))


_NC = 2
_NS = 16
_NW = _NC * _NS
_EB = 128
_BPW = 78
_GB = 6
_GW = _GB * _EB
_NGRP = _BPW // _GB
_NPAIR = 6
_TAILW = _NW - 4
_TAILB = _NW * _BPW
_RPT = 624
_ZR = 16


def _sc_agg(h, ei):
    mesh = plsc.VectorSubcoreMesh(core_axis_name="c", subcore_axis_name="s")

    @functools.partial(
        pl.kernel,
        out_type=jax.ShapeDtypeStruct((_NC, N_NODES, D), jnp.float32),
        mesh=mesh,
        scratch_types=[
            pltpu.VMEM((2, _GW), jnp.int32),
            pltpu.VMEM((2, _GW), jnp.int32),
            pltpu.VMEM((8, _EB), jnp.int32),
            pltpu.VMEM((8, _EB), jnp.int32),
            pltpu.VMEM((_EB, D), jnp.float32),
            pltpu.VMEM((_EB, D), jnp.float32),
            pltpu.VMEM((_ZR, D), jnp.float32),
            pltpu.VMEM_SHARED((N_NODES, D), jnp.float32),
            pltpu.SemaphoreType.DMA,
            pltpu.SemaphoreType.DMA,
            pltpu.SemaphoreType.DMA,
            pltpu.SemaphoreType.DMA,
        ],
    )
    def k(h_hbm, ei_hbm, out_hbm, sgA, sgB, dgA, dgB, rows0_v, rows1_v,
          z_v, acc_s, semA, semB, sem0, sem1):
        c_ax = lax.axis_index("c")
        s = lax.axis_index("s")
        wid = s * _NC + c_ax
        base_e = wid * (_BPW * _EB)
        rows = (rows0_v, rows1_v)
        sems = (sem0, sem1)

        def idx_start(g, sg, sem):
            off = base_e + g * _GW
            pltpu.async_copy(ei_hbm.at[pl.ds(0, 2), pl.ds(off, _GW)], sg, sem)

        def idx_wait(sg, sem):
            pltpu.make_async_copy(ei_hbm.at[pl.ds(0, 2), pl.ds(0, _GW)], sg,
                                  sem).wait()

        def g_start(sg, cc, rbuf, sem):
            pltpu.async_copy(h_hbm.at[sg.at[0, pl.ds(cc * _EB, _EB)]], rbuf,
                             sem)

        def g_wait(sg, cc, rbuf, sem):
            pltpu.make_async_copy(h_hbm.at[sg.at[0, pl.ds(cc * _EB, _EB)]],
                                  rbuf, sem).wait()

        def fill_dg(sg, dg):
            for cc in range(_GB):
                for v in range(_EB // 16):
                    dg[cc, pl.ds(v * 16, 16)] = sg[1,
                                                   pl.ds(cc * _EB + v * 16,
                                                         16)]

        def do_group(sg, dg, chain_fn):
            fill_dg(sg, dg)
            for cc in range(_GB):
                p = cc % 2
                q = (cc + 1) % 2
                if cc + 1 < _GB:
                    g_start(sg, cc + 1, rows[q], sems[q])
                else:
                    chain_fn()
                g_wait(sg, cc, rows[p], sems[p])
                pltpu.sync_copy(rows[p], acc_s.at[dg.at[cc]], add=True)

        idx_start(0, sgA, semA)
        idx_start(1, sgB, semB)

        zv = jnp.zeros((16,), jnp.float32)
        for i in range(_ZR):
            for j in range(D // 16):
                z_v[i, pl.ds(j * 16, 16)] = zv

        def zbody(r, carry):
            pltpu.sync_copy(z_v, acc_s.at[pl.ds(s * _RPT + r * _ZR, _ZR)])
            return carry

        lax.fori_loop(0, _RPT // _ZR, zbody, 0)

        @pl.when(s == _NS - 1)
        def _init_tail():
            pltpu.sync_copy(z_v, acc_s.at[pl.ds(_NS * _RPT, _ZR)])

        idx_wait(sgA, semA)
        g_start(sgA, 0, rows0_v, sem0)

        plsc.subcore_barrier()

        def chainB():
            idx_wait(sgB, semB)
            g_start(sgB, 0, rows0_v, sem0)

        def chainA():
            idx_wait(sgA, semA)
            g_start(sgA, 0, rows0_v, sem0)

        def body(jj, carry):
            do_group(sgA, dgA, chainB)
            idx_start(2 * jj + 2, sgA, semA)
            do_group(sgB, dgB, chainA)

            @pl.when(jj < _NPAIR - 1)
            def _prefetch_b():
                idx_start(2 * jj + 3, sgB, semB)

            return carry

        lax.fori_loop(0, _NPAIR, body, 0)

        do_group(sgA, dgA, lambda: None)

        @pl.when(wid >= _TAILW)
        def _tail_block():
            off = (_TAILB + wid - _TAILW) * _EB
            pltpu.sync_copy(ei_hbm.at[pl.ds(0, 2), pl.ds(off, _EB)],
                            sgA.at[pl.ds(0, 2), pl.ds(0, _EB)])
            for v in range(_EB // 16):
                dgA[0, pl.ds(v * 16, 16)] = sgA[1, pl.ds(v * 16, 16)]
            pltpu.async_copy(h_hbm.at[sgA.at[0, pl.ds(0, _EB)]], rows0_v,
                             sem0).wait()
            pltpu.sync_copy(rows0_v, acc_s.at[dgA.at[0]], add=True)

        plsc.subcore_barrier()

        pltpu.sync_copy(acc_s.at[pl.ds(s * _RPT, _RPT)],
                        out_hbm.at[c_ax, pl.ds(s * _RPT, _RPT)])

        @pl.when(s == _NS - 1)
        def _out_tail():
            pltpu.sync_copy(acc_s.at[pl.ds(_NS * _RPT, _ZR)],
                            out_hbm.at[c_ax, pl.ds(_NS * _RPT, _ZR)])

    return k(h, ei)


def _mlp2_body(h_ref, n0_ref, n1_ref, w_ref, b_ref, eps_ref, o_ref):
    t = 1.0 + eps_ref[0, 0] * h_ref[...] + n0_ref[0] + n1_ref[0]
    y = lax.dot_general(t, w_ref[...], (((1,), (1,)), ((), ())),
                        preferred_element_type=jnp.float32)
    o_ref[...] = jnp.maximum(y + b_ref[...], 0.0)


def _mlp2(h, n_parts, W_phy, b_phy, eps):
    blk = lambda i: (i, 0)
    return pl.pallas_call(
        _mlp2_body,
        grid=(N_NODES // _ROWS,),
        in_specs=[
            pl.BlockSpec((_ROWS, D), blk),
            pl.BlockSpec((1, _ROWS, D), lambda i: (0, i, 0)),
            pl.BlockSpec((1, _ROWS, D), lambda i: (1, i, 0)),
            pl.BlockSpec((D, D), lambda i: (0, 0)),
            pl.BlockSpec((1, D), lambda i: (0, 0)),
            pl.BlockSpec((1, 1), lambda i: (0, 0)),
        ],
        out_specs=pl.BlockSpec((_ROWS, D), blk),
        out_shape=jax.ShapeDtypeStruct((N_NODES, D), jnp.float32),
    )(h, n_parts, n_parts, W_phy, b_phy.reshape(1, D), eps.reshape(1, 1))


def kernel(feats, edge_index, W_f, b_f, W_phy, b_phy, eps):
    ei = edge_index.astype(jnp.int32)
    h = _mlp1(feats, W_f, b_f)
    n_parts = _sc_agg(h, ei)
    return _mlp2(h, n_parts, W_phy, b_phy, eps)
